# baseline (speedup 1.0000x reference)
import jax
import jax.numpy as jnp
from jax import lax
from jax.experimental import pallas as pl
from jax.experimental.pallas import tpu as pltpu
from jax.experimental.pallas import tpu_sc as plsc

NC = 2
NS = 16
NW = NC * NS
NBUF = 4
LANE = 128
SUB = 8


def _body(x_hbm, table2_hbm, out_hbm, idx_v, table_v, table_sh, bufs,
          gsems, wsems):
    sid = lax.axis_index("s")
    wid = sid * NC + lax.axis_index("c")
    cperw = x_hbm.shape[1]
    nslot = x_hbm.shape[2]

    @pl.when(sid == 0)
    def _():
        pltpu.sync_copy(table2_hbm, table_v)
        pltpu.sync_copy(table_v, table_sh)

    pltpu.sync_copy(x_hbm.at[wid], idx_v)
    plsc.subcore_barrier()

    def gather(c, b):
        pltpu.make_async_copy(
            table_sh.at[idx_v.at[c]], bufs[b], gsems[b]).start()

    def wait_gather(b):
        pltpu.make_async_copy(
            table_sh.at[idx_v.at[0]], bufs[b], gsems[b]).wait()

    def write(c, b):
        pltpu.make_async_copy(
            bufs[b], out_hbm.at[wid * cperw + c, pl.ds(0, nslot)],
            wsems[b]).start()

    def wait_write(b):
        pltpu.make_async_copy(
            bufs[b], out_hbm.at[0, pl.ds(0, nslot)], wsems[b]).wait()

    for b in range(NBUF):
        gather(b, b)

    def step(g, carry):
        c0 = NBUF * g
        for b in range(NBUF):
            wait_gather(b)
            write(c0 + b, b)
        for b in range(NBUF):
            wait_write(b)
            gather(c0 + NBUF + b, b)
        return carry

    lax.fori_loop(0, cperw // NBUF - 1, step, 0)

    c0 = cperw - NBUF
    for b in range(NBUF):
        wait_gather(b)
        write(c0 + b, b)
    for b in range(NBUF):
        wait_write(b)


def kernel(x, table):
    B0, B1 = x.shape
    V, D = table.shape
    cperw = B0 // NW
    rows_p = ((B1 + SUB - 1) // SUB) * SUB
    cols_p = ((D + LANE - 1) // LANE) * LANE

    table2 = jnp.pad(table, ((0, 0), (0, cols_p - D)))
    x3 = x.reshape(NW, cperw, B1)

    fn = pl.kernel(
        _body,
        out_type=jax.ShapeDtypeStruct((B0, rows_p, cols_p), jnp.float32),
        mesh=plsc.VectorSubcoreMesh(core_axis_name="c", subcore_axis_name="s"),
        compiler_params=pltpu.CompilerParams(use_tc_tiling_on_sc=False),
        scratch_types=[
            pltpu.VMEM((cperw, B1), jnp.int32),
            pltpu.VMEM((V, cols_p), jnp.float32),
            pltpu.VMEM_SHARED((V, cols_p), jnp.float32),
            [pltpu.VMEM((B1, cols_p), jnp.float32) for _ in range(NBUF)],
            [pltpu.SemaphoreType.DMA for _ in range(NBUF)],
            [pltpu.SemaphoreType.DMA for _ in range(NBUF)],
        ],
    )
    padded = fn(x3, table2)
    return padded[:, :B1, :D]


# speedup vs baseline: 1.0348x; 1.0348x over previous
import jax
import jax.numpy as jnp
import numpy as np
from jax import lax
from jax.experimental import pallas as pl
from jax.experimental.pallas import tpu as pltpu
from jax.experimental.pallas import tpu_sc as plsc

NC = 2
NS = 16
NW = NC * NS
NBUF = 4
LANE = 128
SUB = 8
CB = 32


def _body(idx2_hbm, table2_hbm, pat_hbm, out_hbm, idx_v, pat_v, table_v,
          table_sh, bufs, gsems, wsems):
    sid = lax.axis_index("s")
    wid = sid * NC + lax.axis_index("c")
    cperw = idx2_hbm.shape[1]

    @pl.when(sid == 0)
    def _():
        pltpu.sync_copy(table2_hbm, table_v)
        pltpu.sync_copy(table_v, table_sh)

    pltpu.sync_copy(idx2_hbm.at[wid], idx_v)
    pltpu.sync_copy(pat_hbm, pat_v)
    plsc.subcore_barrier()

    def gather(c, b):
        pltpu.make_async_copy(
            table_sh.at[idx_v.at[c]], bufs[b], gsems[b]).start()

    def wait_gather(b):
        pltpu.make_async_copy(
            table_sh.at[idx_v.at[0]], bufs[b], gsems[b]).wait()

    def scatter(c, b):
        pltpu.make_async_copy(
            bufs[b], out_hbm.at[wid * cperw + c].at[pat_v.at[0]],
            wsems[b]).start()

    def wait_scatter(b):
        pltpu.make_async_copy(
            bufs[b], out_hbm.at[0].at[pat_v.at[0]], wsems[b]).wait()

    for b in range(NBUF):
        gather(b, b)

    def step(g, carry):
        c0 = NBUF * g
        for b in range(NBUF):
            wait_gather(b)
            scatter(c0 + b, b)
        for b in range(NBUF):
            wait_scatter(b)
            gather(c0 + NBUF + b, b)
        return carry

    lax.fori_loop(0, cperw // NBUF - 1, step, 0)

    c0 = cperw - NBUF
    for b in range(NBUF):
        wait_gather(b)
        scatter(c0 + b, b)
    for b in range(NBUF):
        wait_scatter(b)


def _convert_body(t_ref, o_ref):
    B1, D = o_ref.shape[1], o_ref.shape[2]
    jt_n = (B1 + SUB - 1) // SUB
    for jt in range(jt_n):
        rmax = min(SUB, B1 - jt * SUB)
        for half in range(2):
            cmax = min(LANE, D - half * LANE)
            src_r = (jt * 2 + half) * SUB
            o_ref[:, jt * SUB:jt * SUB + rmax,
                  half * LANE:half * LANE + cmax] = (
                t_ref[:, src_r:src_r + rmax, :cmax])


def kernel(x, table):
    B0, B1 = x.shape
    V, D = table.shape
    cperw = B0 // NW
    jt = (B1 + SUB - 1) // SUB
    dt = (D + LANE - 1) // LANE
    nseg = B1 * dt

    table2 = jnp.pad(table, ((0, 0), (0, dt * LANE - D))).reshape(
        V * dt, LANE)

    x3 = x.reshape(NW, cperw, B1)
    idx2 = jnp.stack([2 * x3, 2 * x3 + 1], axis=-1).reshape(NW, cperw, nseg)

    j = np.arange(B1)
    base = (j // SUB) * (dt * SUB) + (j % SUB)
    pat = np.stack([base, base + SUB], axis=-1).reshape(1, nseg)
    pat = jnp.asarray(pat, dtype=jnp.int32)

    fn = pl.kernel(
        _body,
        out_type=jax.ShapeDtypeStruct((B0, jt * dt * SUB, LANE), jnp.float32),
        mesh=plsc.VectorSubcoreMesh(core_axis_name="c", subcore_axis_name="s"),
        compiler_params=pltpu.CompilerParams(use_tc_tiling_on_sc=False),
        scratch_types=[
            pltpu.VMEM((cperw, nseg), jnp.int32),
            pltpu.VMEM((1, nseg), jnp.int32),
            pltpu.VMEM((V * dt, LANE), jnp.float32),
            pltpu.VMEM_SHARED((V * dt, LANE), jnp.float32),
            [pltpu.VMEM((nseg, LANE), jnp.float32) for _ in range(NBUF)],
            [pltpu.SemaphoreType.DMA for _ in range(NBUF)],
            [pltpu.SemaphoreType.DMA for _ in range(NBUF)],
        ],
    )
    t5 = fn(idx2, table2, pat)

    conv = pl.pallas_call(
        _convert_body,
        grid=(B0 // CB,),
        in_specs=[pl.BlockSpec((CB, jt * dt * SUB, LANE),
                               lambda g: (g, 0, 0))],
        out_specs=pl.BlockSpec((CB, B1, D), lambda g: (g, 0, 0)),
        out_shape=jax.ShapeDtypeStruct((B0, B1, D), jnp.float32),
    )
    return conv(t5)


# speedup vs baseline: 1.9329x; 1.8679x over previous
import jax
import jax.numpy as jnp
import numpy as np
from jax import lax
from jax.experimental import pallas as pl
from jax.experimental.pallas import tpu as pltpu
from jax.experimental.pallas import tpu_sc as plsc

NC = 2
NS = 16
NW = NC * NS
NBUF = 4
LANE = 128
SUB = 8
IB = 128


def _pat_row(j, half, B1):
    jt, js = j // SUB, j % SUB
    full = min(SUB, B1 - jt * SUB)
    base = 2 * SUB * jt if full == SUB else 2 * SUB * jt
    return base + half * full + js


def _body(idx2_hbm, table2_hbm, pat_hbm, out_hbm, idx_v, pat_v, table_v,
          table_sh, bufs, gsems, wsems):
    sid = lax.axis_index("s")
    wid = sid * NC + lax.axis_index("c")
    cperw = idx2_hbm.shape[1]

    @pl.when(sid == 0)
    def _():
        pltpu.sync_copy(table2_hbm, table_v)
        pltpu.sync_copy(table_v, table_sh)

    pltpu.sync_copy(idx2_hbm.at[wid], idx_v)
    pltpu.sync_copy(pat_hbm, pat_v)
    plsc.subcore_barrier()

    def gather(c, b):
        pltpu.make_async_copy(
            table_sh.at[idx_v.at[c]], bufs[b], gsems[b]).start()

    def wait_gather(b):
        pltpu.make_async_copy(
            table_sh.at[idx_v.at[0]], bufs[b], gsems[b]).wait()

    def scatter(c, b):
        pltpu.make_async_copy(
            bufs[b], out_hbm.at[wid * cperw + c].at[pat_v.at[0]],
            wsems[b]).start()

    def wait_scatter(b):
        pltpu.make_async_copy(
            bufs[b], out_hbm.at[0].at[pat_v.at[0]], wsems[b]).wait()

    for b in range(NBUF):
        gather(b, b)

    def step(g, carry):
        c0 = NBUF * g
        for b in range(NBUF):
            wait_gather(b)
            scatter(c0 + b, b)
        for b in range(NBUF):
            wait_scatter(b)
            gather(c0 + NBUF + b, b)
        return carry

    lax.fori_loop(0, cperw // NBUF - 1, step, 0)

    c0 = cperw - NBUF
    for b in range(NBUF):
        wait_gather(b)
        scatter(c0 + b, b)
    for b in range(NBUF):
        wait_scatter(b)


def _make_convert_body(B1, D):
    def _convert_body(t_ref, o_ref):
        eye = (lax.broadcasted_iota(jnp.int32, (IB, IB), 0)
               == lax.broadcasted_iota(jnp.int32, (IB, IB), 1)
               ).astype(jnp.float32)
        for j in range(B1):
            for half in range(2):
                cmax = min(LANE, D - half * LANE)
                row = _pat_row(j, half, B1)
                sheet = t_ref[:, row, :]
                tr = lax.dot_general(
                    sheet, eye, (((0,), (0,)), ((), ())),
                    preferred_element_type=jnp.float32)
                o_ref[j, half * LANE:half * LANE + cmax, :] = tr[:cmax, :]
    return _convert_body


def kernel(x, table):
    B0, B1 = x.shape
    V, D = table.shape
    cperw = B0 // NW
    dt = (D + LANE - 1) // LANE
    nseg = B1 * dt

    table2 = jnp.pad(table, ((0, 0), (0, dt * LANE - D))).reshape(
        V * dt, LANE)

    x3 = x.reshape(NW, cperw, B1)
    idx2 = jnp.stack([2 * x3, 2 * x3 + 1], axis=-1).reshape(NW, cperw, nseg)

    pat = np.zeros((1, nseg), dtype=np.int32)
    for j in range(B1):
        for half in range(2):
            pat[0, 2 * j + half] = _pat_row(j, half, B1)
    pat = jnp.asarray(pat)

    nrow = ((nseg + SUB - 1) // SUB) * SUB
    fn = pl.kernel(
        _body,
        out_type=jax.ShapeDtypeStruct((B0, nrow, LANE), jnp.float32),
        mesh=plsc.VectorSubcoreMesh(core_axis_name="c", subcore_axis_name="s"),
        compiler_params=pltpu.CompilerParams(use_tc_tiling_on_sc=False),
        scratch_types=[
            pltpu.VMEM((cperw, nseg), jnp.int32),
            pltpu.VMEM((1, nseg), jnp.int32),
            pltpu.VMEM((V * dt, LANE), jnp.float32),
            pltpu.VMEM_SHARED((V * dt, LANE), jnp.float32),
            [pltpu.VMEM((nseg, LANE), jnp.float32) for _ in range(NBUF)],
            [pltpu.SemaphoreType.DMA for _ in range(NBUF)],
            [pltpu.SemaphoreType.DMA for _ in range(NBUF)],
        ],
    )
    t5 = fn(idx2, table2, pat)

    conv = pl.pallas_call(
        _make_convert_body(B1, D),
        grid=(B0 // IB,),
        in_specs=[pl.BlockSpec((IB, nrow, LANE), lambda g: (g, 0, 0))],
        out_specs=pl.BlockSpec((B1, D, IB), lambda g: (0, 0, g)),
        out_shape=jax.ShapeDtypeStruct((B1, D, B0), jnp.float32),
    )
    out_t = conv(t5)
    return jnp.transpose(out_t, (2, 0, 1))


# speedup vs baseline: 1.9535x; 1.0107x over previous
import jax
import jax.numpy as jnp
import numpy as np
from jax import lax
from jax.experimental import pallas as pl
from jax.experimental.pallas import tpu as pltpu
from jax.experimental.pallas import tpu_sc as plsc

NC = 2
NS = 16
NW = NC * NS
NBUF = 4
LANE = 128
SUB = 8
IB = 128


def _pat_row(j, half, B1):
    jt, js = j // SUB, j % SUB
    full = min(SUB, B1 - jt * SUB)
    base = 2 * SUB * jt if full == SUB else 2 * SUB * jt
    return base + half * full + js


def _body(idx2_hbm, table2_hbm, pat_hbm, out_hbm, idx_v, pat_v, table_v,
          table_sh, bufs, gsems, wsems):
    sid = lax.axis_index("s")
    wid = sid * NC + lax.axis_index("c")
    cperw = idx2_hbm.shape[1]

    @pl.when(sid == 0)
    def _():
        pltpu.sync_copy(table2_hbm, table_v)
        pltpu.sync_copy(table_v, table_sh)

    pltpu.sync_copy(idx2_hbm.at[wid], idx_v)
    pltpu.sync_copy(pat_hbm, pat_v)
    plsc.subcore_barrier()

    def gather(c, b):
        pltpu.make_async_copy(
            table_sh.at[idx_v.at[c]], bufs[b], gsems[b]).start()

    def wait_gather(b):
        pltpu.make_async_copy(
            table_sh.at[idx_v.at[0]], bufs[b], gsems[b]).wait()

    def scatter(c, b):
        pltpu.make_async_copy(
            bufs[b], out_hbm.at[wid * cperw + c].at[pat_v.at[0]],
            wsems[b]).start()

    def wait_scatter(b):
        pltpu.make_async_copy(
            bufs[b], out_hbm.at[0].at[pat_v.at[0]], wsems[b]).wait()

    for b in range(NBUF):
        gather(b, b)

    def step(g, carry):
        c0 = NBUF * g
        for b in range(NBUF):
            wait_gather(b)
            scatter(c0 + b, b)
        for b in range(NBUF):
            wait_scatter(b)
            gather(c0 + NBUF + b, b)
        return carry

    lax.fori_loop(0, cperw // NBUF - 1, step, 0)

    c0 = cperw - NBUF
    for b in range(NBUF):
        wait_gather(b)
        scatter(c0 + b, b)
    for b in range(NBUF):
        wait_scatter(b)


def _make_convert_body(B1, D):
    def _convert_body(t_ref, o_ref):
        for j in range(B1):
            for half in range(2):
                cmax = min(LANE, D - half * LANE)
                row = _pat_row(j, half, B1)
                sheet = t_ref[:, row, :]
                tr = jnp.transpose(sheet)
                o_ref[j, half * LANE:half * LANE + cmax, :] = tr[:cmax, :]
    return _convert_body


def kernel(x, table):
    B0, B1 = x.shape
    V, D = table.shape
    cperw = B0 // NW
    dt = (D + LANE - 1) // LANE
    nseg = B1 * dt

    table2 = jnp.pad(table, ((0, 0), (0, dt * LANE - D))).reshape(
        V * dt, LANE)

    x3 = x.reshape(NW, cperw, B1)
    idx2 = jnp.stack([2 * x3, 2 * x3 + 1], axis=-1).reshape(NW, cperw, nseg)

    pat = np.zeros((1, nseg), dtype=np.int32)
    for j in range(B1):
        for half in range(2):
            pat[0, 2 * j + half] = _pat_row(j, half, B1)
    pat = jnp.asarray(pat)

    nrow = ((nseg + SUB - 1) // SUB) * SUB
    fn = pl.kernel(
        _body,
        out_type=jax.ShapeDtypeStruct((B0, nrow, LANE), jnp.float32),
        mesh=plsc.VectorSubcoreMesh(core_axis_name="c", subcore_axis_name="s"),
        compiler_params=pltpu.CompilerParams(use_tc_tiling_on_sc=False),
        scratch_types=[
            pltpu.VMEM((cperw, nseg), jnp.int32),
            pltpu.VMEM((1, nseg), jnp.int32),
            pltpu.VMEM((V * dt, LANE), jnp.float32),
            pltpu.VMEM_SHARED((V * dt, LANE), jnp.float32),
            [pltpu.VMEM((nseg, LANE), jnp.float32) for _ in range(NBUF)],
            [pltpu.SemaphoreType.DMA for _ in range(NBUF)],
            [pltpu.SemaphoreType.DMA for _ in range(NBUF)],
        ],
    )
    t5 = fn(idx2, table2, pat)

    conv = pl.pallas_call(
        _make_convert_body(B1, D),
        grid=(B0 // IB,),
        in_specs=[pl.BlockSpec((IB, nrow, LANE), lambda g: (g, 0, 0))],
        out_specs=pl.BlockSpec((B1, D, IB), lambda g: (0, 0, g)),
        out_shape=jax.ShapeDtypeStruct((B1, D, B0), jnp.float32),
    )
    out_t = conv(t5)
    return jnp.transpose(out_t, (2, 0, 1))


# speedup vs baseline: 2.0058x; 1.0267x over previous
import jax
import jax.numpy as jnp
import numpy as np
from jax import lax
from jax.experimental import pallas as pl
from jax.experimental.pallas import tpu as pltpu
from jax.experimental.pallas import tpu_sc as plsc

NC = 2
NS = 16
NW = NC * NS
NBUF = 4
LANE = 128
SUB = 8
IB = 128


def _pat_row(j, half, B1):
    jt, js = j // SUB, j % SUB
    full = min(SUB, B1 - jt * SUB)
    base = 2 * SUB * jt if full == SUB else 2 * SUB * jt
    return base + half * full + js


def _body(idx2_hbm, table2_hbm, pat_hbm, out_hbm, idx_v, pat_v, table_v,
          table_sh, bufs, gsems, wsems):
    sid = lax.axis_index("s")
    wid = sid * NC + lax.axis_index("c")
    cperw = idx2_hbm.shape[1]

    @pl.when(sid == 0)
    def _():
        pltpu.sync_copy(table2_hbm, table_v)
        pltpu.sync_copy(table_v, table_sh)

    pltpu.sync_copy(idx2_hbm.at[wid], idx_v)
    pltpu.sync_copy(pat_hbm, pat_v)
    plsc.subcore_barrier()

    def gather(c, b):
        pltpu.make_async_copy(
            table_sh.at[idx_v.at[c]], bufs[b], gsems[b]).start()

    def wait_gather(b):
        pltpu.make_async_copy(
            table_sh.at[idx_v.at[0]], bufs[b], gsems[b]).wait()

    def scatter(c, b):
        pltpu.make_async_copy(
            bufs[b], out_hbm.at[wid * cperw + c].at[pat_v.at[0]],
            wsems[b]).start()

    def wait_scatter(b):
        pltpu.make_async_copy(
            bufs[b], out_hbm.at[0].at[pat_v.at[0]], wsems[b]).wait()

    for b in range(NBUF):
        gather(b, b)

    def step(g, carry):
        c0 = NBUF * g
        for b in range(NBUF):
            wait_gather(b)
            scatter(c0 + b, b)
        for b in range(NBUF):
            wait_scatter(b)
            gather(c0 + NBUF + b, b)
        return carry

    lax.fori_loop(0, cperw // NBUF - 1, step, 0)

    c0 = cperw - NBUF
    for b in range(NBUF):
        wait_gather(b)
        scatter(c0 + b, b)
    for b in range(NBUF):
        wait_scatter(b)


def _make_convert_body(B1, D):
    def _convert_body(t_ref, o_ref):
        for j in range(B1):
            for half in range(2):
                cmax = min(LANE, D - half * LANE)
                row = _pat_row(j, half, B1)
                sheet = t_ref[:, row, :]
                tr = jnp.transpose(sheet)
                o_ref[j, half * LANE:half * LANE + cmax, :] = tr[:cmax, :]
    return _convert_body


def kernel(x, table):
    B0, B1 = x.shape
    V, D = table.shape
    half_b = B0 // 2
    cperw = half_b // NW
    dt = (D + LANE - 1) // LANE
    nseg = B1 * dt
    nrow = ((nseg + SUB - 1) // SUB) * SUB

    table2 = jnp.pad(table, ((0, 0), (0, dt * LANE - D))).reshape(
        V * dt, LANE)

    x4 = x.reshape(2, NW, cperw, B1)
    idx2 = jnp.stack([2 * x4, 2 * x4 + 1], axis=-1).reshape(
        2, NW, cperw, nseg)

    pat = np.zeros((1, nseg), dtype=np.int32)
    for j in range(B1):
        for half in range(2):
            pat[0, 2 * j + half] = _pat_row(j, half, B1)
    pat = jnp.asarray(pat)

    def sc_gather(idx2_half):
        fn = pl.kernel(
            _body,
            out_type=jax.ShapeDtypeStruct((half_b, nrow, LANE), jnp.float32),
            mesh=plsc.VectorSubcoreMesh(core_axis_name="c",
                                        subcore_axis_name="s"),
            compiler_params=pltpu.CompilerParams(use_tc_tiling_on_sc=False),
            scratch_types=[
                pltpu.VMEM((cperw, nseg), jnp.int32),
                pltpu.VMEM((1, nseg), jnp.int32),
                pltpu.VMEM((V * dt, LANE), jnp.float32),
                pltpu.VMEM_SHARED((V * dt, LANE), jnp.float32),
                [pltpu.VMEM((nseg, LANE), jnp.float32) for _ in range(NBUF)],
                [pltpu.SemaphoreType.DMA for _ in range(NBUF)],
                [pltpu.SemaphoreType.DMA for _ in range(NBUF)],
            ],
        )
        return fn(idx2_half, table2, pat)

    grid_half = half_b // IB
    body = _make_convert_body(B1, D)

    t5a = sc_gather(idx2[0])
    t5b = sc_gather(idx2[1])

    conv_a = pl.pallas_call(
        body,
        grid=(grid_half,),
        in_specs=[pl.BlockSpec((IB, nrow, LANE), lambda g: (g, 0, 0))],
        out_specs=pl.BlockSpec((B1, D, IB), lambda g: (0, 0, g)),
        out_shape=jax.ShapeDtypeStruct((B1, D, B0), jnp.float32),
    )
    out1 = conv_a(t5a)

    def body_b(t_ref, _prev, o_ref):
        body(t_ref, o_ref)

    conv_b = pl.pallas_call(
        body_b,
        grid=(grid_half,),
        in_specs=[pl.BlockSpec((IB, nrow, LANE), lambda g: (g, 0, 0)),
                  pl.BlockSpec(memory_space=pl.ANY)],
        out_specs=pl.BlockSpec((B1, D, IB),
                               lambda g: (0, 0, g + grid_half)),
        out_shape=jax.ShapeDtypeStruct((B1, D, B0), jnp.float32),
        input_output_aliases={1: 0},
    )
    out_t = conv_b(t5b, out1)
    return jnp.transpose(out_t, (2, 0, 1))


# speedup vs baseline: 2.0108x; 1.0025x over previous
import jax
import jax.numpy as jnp
import numpy as np
from jax import lax
from jax.experimental import pallas as pl
from jax.experimental.pallas import tpu as pltpu
from jax.experimental.pallas import tpu_sc as plsc

NC = 2
NS = 16
NW = NC * NS
NBUF = 4
LANE = 128
SUB = 8
IB = 128


def _pat_row(j, half, B1):
    jt, js = j // SUB, j % SUB
    full = min(SUB, B1 - jt * SUB)
    return 2 * SUB * jt + half * full + js


def _body(idx2_hbm, table2_hbm, pat_hbm, out_hbm, idx_v, pat_v, table_v,
          table_sh, bufs, gsems, wsems):
    sid = lax.axis_index("s")
    wid = sid * NC + lax.axis_index("c")
    cperw = idx2_hbm.shape[1]

    @pl.when(sid == 0)
    def _():
        pltpu.sync_copy(table2_hbm, table_v)
        pltpu.sync_copy(table_v, table_sh)

    pltpu.sync_copy(idx2_hbm.at[wid], idx_v)
    pltpu.sync_copy(pat_hbm, pat_v)
    plsc.subcore_barrier()

    def gather(c, b):
        pltpu.make_async_copy(
            table_sh.at[idx_v.at[c]], bufs[b], gsems[b]).start()

    def wait_gather(b):
        pltpu.make_async_copy(
            table_sh.at[idx_v.at[0]], bufs[b], gsems[b]).wait()

    def scatter(c, b):
        pltpu.make_async_copy(
            bufs[b], out_hbm.at[wid * cperw + c].at[pat_v.at[0]],
            wsems[b]).start()

    def wait_scatter(b):
        pltpu.make_async_copy(
            bufs[b], out_hbm.at[0].at[pat_v.at[0]], wsems[b]).wait()

    for b in range(NBUF):
        gather(b, b)

    def step(g, carry):
        c0 = NBUF * g
        for b in range(NBUF):
            wait_gather(b)
            scatter(c0 + b, b)
        for b in range(NBUF):
            wait_scatter(b)
            gather(c0 + NBUF + b, b)
        return carry

    lax.fori_loop(0, cperw // NBUF - 1, step, 0)

    c0 = cperw - NBUF
    for b in range(NBUF):
        wait_gather(b)
        scatter(c0 + b, b)
    for b in range(NBUF):
        wait_scatter(b)


def _make_convert_body(B1, D):
    def _convert_body(t_ref, o_ref):
        for j in range(B1):
            for half in range(2):
                cmax = min(LANE, D - half * LANE)
                row = _pat_row(j, half, B1)
                sheet = t_ref[:, row, :]
                tr = jnp.transpose(sheet)
                o_ref[j, half * LANE:half * LANE + cmax, :] = tr[:cmax, :]
    return _convert_body


def kernel(x, table):
    B0, B1 = x.shape
    V, D = table.shape
    half_b = B0 // 2
    cperw = half_b // NW
    dt = (D + LANE - 1) // LANE
    nseg = B1 * dt
    nrow = ((nseg + SUB - 1) // SUB) * SUB

    table2 = jnp.pad(table, ((0, 0), (0, dt * LANE - D))).reshape(
        V * dt, LANE)

    x4 = x.reshape(2, NW, cperw, B1)
    idx2 = jnp.stack([2 * x4, 2 * x4 + 1], axis=-1).reshape(
        2, NW, cperw, nseg)

    pat = np.zeros((1, nseg), dtype=np.int32)
    for j in range(B1):
        for half in range(2):
            pat[0, 2 * j + half] = _pat_row(j, half, B1)
    pat = jnp.asarray(pat)

    def sc_gather(idx2_half):
        fn = pl.kernel(
            _body,
            out_type=jax.ShapeDtypeStruct((half_b, nrow, LANE), jnp.float32),
            mesh=plsc.VectorSubcoreMesh(core_axis_name="c",
                                        subcore_axis_name="s"),
            compiler_params=pltpu.CompilerParams(use_tc_tiling_on_sc=False),
            scratch_types=[
                pltpu.VMEM((cperw, nseg), jnp.int32),
                pltpu.VMEM((1, nseg), jnp.int32),
                pltpu.VMEM((V * dt, LANE), jnp.float32),
                pltpu.VMEM_SHARED((V * dt, LANE), jnp.float32),
                [pltpu.VMEM((nseg, LANE), jnp.float32) for _ in range(NBUF)],
                [pltpu.SemaphoreType.DMA for _ in range(NBUF)],
                [pltpu.SemaphoreType.DMA for _ in range(NBUF)],
            ],
        )
        return fn(idx2_half, table2, pat)

    grid_half = half_b // IB
    body = _make_convert_body(B1, D)

    t5a = sc_gather(idx2[0])
    t5b = sc_gather(idx2[1])

    conv_a = pl.pallas_call(
        body,
        grid=(grid_half,),
        in_specs=[pl.BlockSpec((IB, nrow, LANE), lambda g: (g, 0, 0))],
        out_specs=pl.BlockSpec((B1, D, IB), lambda g: (0, 0, g)),
        out_shape=jax.ShapeDtypeStruct((B1, D, B0), jnp.float32),
    )
    out1 = conv_a(t5a)

    def body_b(t_ref, _prev, o_ref):
        body(t_ref, o_ref)

    conv_b = pl.pallas_call(
        body_b,
        grid=(grid_half,),
        in_specs=[pl.BlockSpec((IB, nrow, LANE), lambda g: (g, 0, 0)),
                  pl.BlockSpec(memory_space=pl.ANY)],
        out_specs=pl.BlockSpec((B1, D, IB),
                               lambda g: (0, 0, g + grid_half)),
        out_shape=jax.ShapeDtypeStruct((B1, D, B0), jnp.float32),
        input_output_aliases={1: 0},
    )
    out_t = conv_b(t5b, out1)
    return jnp.transpose(out_t, (2, 0, 1))
